# baseline (speedup 1.0000x reference)
import functools

import jax
import jax.numpy as jnp
from jax import lax
from jax.experimental import pallas as pl
from jax.experimental.pallas import tpu as pltpu
from jax.experimental.pallas import tpu_sc as plsc

_B, _S, _D, _F, _E, _K = 2, 2048, 1024, 1024, 64, 2
_T = _B * _S
_P = _T * _K
_BT = 128
_NB = _P // _BT + _E
_G = _NB * _BT
_TB = 64
_TA = 256

_NC, _NS = 2, 16
_NW = _NC * _NS


def _router_body(x_ref, rw_ref, alpha_ref, e0_ref, e1_ref, wa0_ref, wa1_ref,
                 r0_ref, r1_ref, c_ref, poff_ref, run_ref):
    i = pl.program_id(0)

    @pl.when(i == 0)
    def _():
        run_ref[...] = jnp.zeros_like(run_ref)

    x = x_ref[...]
    logits = lax.dot_general(x, rw_ref[...], (((1,), (1,)), ((), ())),
                             preferred_element_type=jnp.float32)
    eidx = lax.broadcasted_iota(jnp.int32, (_TB, _E), 1)
    m1 = jnp.max(logits, axis=1)
    a1 = jnp.argmax(logits, axis=1).astype(jnp.int32)
    masked = jnp.where(eidx == a1[:, None], -1e30, logits)
    m2 = jnp.max(masked, axis=1)
    a2 = jnp.argmax(masked, axis=1).astype(jnp.int32)
    w1 = 1.0 / (1.0 + jnp.exp(m2 - m1))
    w2 = 1.0 - w1

    oh1 = (eidx == a1[:, None]).astype(jnp.float32)
    oh2 = (eidx == a2[:, None]).astype(jnp.float32)
    al = alpha_ref[0, :]
    as1 = jnp.sum(oh1 * al[None, :], axis=1)
    as2 = jnp.sum(oh2 * al[None, :], axis=1)
    c = 1.0 - (w1 * as1 + w2 * as2)

    oh = jnp.concatenate([oh1, oh2], axis=0)
    t = oh
    for d in (1, 2, 4, 8, 16, 32, 64):
        t = t + jnp.concatenate(
            [jnp.zeros((d, _E), jnp.float32), t[:-d]], axis=0)
    ex = t - oh
    run = run_ref[...]
    rank = ex + run
    rank_vec = jnp.sum(rank * oh, axis=1)
    run_new = run + t[2 * _TB - 1:2 * _TB, :]
    run_ref[...] = run_new

    pad = jnp.floor((run_new + (_BT - 1)) * (1.0 / _BT)) * _BT
    pt = pad
    for d in (1, 2, 4, 8, 16, 32):
        pt = pt + jnp.concatenate(
            [jnp.zeros((1, d), jnp.float32), pt[:, :_E - d]], axis=1)
    poff_ref[...] = pt - pad

    e0_ref[...] = a1.reshape(1, 1, _TB)
    e1_ref[...] = a2.reshape(1, 1, _TB)
    wa0_ref[...] = (w1 * as1).reshape(1, 1, _TB)
    wa1_ref[...] = (w2 * as2).reshape(1, 1, _TB)
    r0_ref[...] = rank_vec[:_TB].reshape(1, 1, _TB)
    r1_ref[...] = rank_vec[_TB:].reshape(1, 1, _TB)
    c_ref[...] = c.reshape(1, 1, _TB)


def _run_router(x_flat, router_weight, alpha2):
    n = _T // _TB
    out3 = jax.ShapeDtypeStruct((n, 1, _TB), jnp.float32)
    out3i = jax.ShapeDtypeStruct((n, 1, _TB), jnp.int32)
    spec3 = pl.BlockSpec((1, 1, _TB), lambda i: (i, 0, 0))
    return pl.pallas_call(
        _router_body,
        grid=(n,),
        in_specs=[
            pl.BlockSpec((_TB, _D), lambda i: (i, 0)),
            pl.BlockSpec((_E, _D), lambda i: (0, 0)),
            pl.BlockSpec((1, _E), lambda i: (0, 0)),
        ],
        out_specs=[spec3, spec3, spec3, spec3, spec3, spec3, spec3,
                   pl.BlockSpec((1, _E), lambda i: (0, 0))],
        out_shape=[out3i, out3i, out3, out3, out3, out3, out3,
                   jax.ShapeDtypeStruct((1, _E), jnp.float32)],
        scratch_shapes=[pltpu.VMEM((1, _E), jnp.float32)],
    )(x_flat, router_weight, alpha2)


def _shared_body(x_ref, gw_ref, uw_ref, dw_ref, c_ref, out_ref):
    x = x_ref[...]
    g = lax.dot_general(x, gw_ref[...], (((1,), (1,)), ((), ())),
                        preferred_element_type=jnp.float32)
    u = lax.dot_general(x, uw_ref[...], (((1,), (1,)), ((), ())),
                        preferred_element_type=jnp.float32)
    h = (g * jax.nn.sigmoid(g) * u).astype(jnp.bfloat16)
    y = lax.dot_general(h, dw_ref[...], (((1,), (1,)), ((), ())),
                        preferred_element_type=jnp.float32)
    out_ref[...] = y * c_ref[0, 0, :][:, None]


def _run_shared(x_flat, sgw, suw, sdw, c3):
    n = _T // _TA
    return pl.pallas_call(
        _shared_body,
        grid=(n,),
        in_specs=[
            pl.BlockSpec((_TA, _D), lambda i: (i, 0)),
            pl.BlockSpec((_F, _D), lambda i: (0, 0)),
            pl.BlockSpec((_F, _D), lambda i: (0, 0)),
            pl.BlockSpec((_D, _F), lambda i: (0, 0)),
            pl.BlockSpec((1, 1, _TA), lambda i: (i, 0, 0)),
        ],
        out_specs=pl.BlockSpec((_TA, _D), lambda i: (i, 0)),
        out_shape=jax.ShapeDtypeStruct((_T, _D), jnp.float32),
    )(x_flat, sgw, suw, sdw, c3)


@functools.lru_cache(maxsize=None)
def _make_route_build():
    mesh = plsc.VectorSubcoreMesh(core_axis_name="c", subcore_axis_name="s", num_cores=_NC, num_subcores=_NS)
    out_type = [
        jax.ShapeDtypeStruct((_G,), jnp.int32),
        jax.ShapeDtypeStruct((_G,), jnp.float32),
        jax.ShapeDtypeStruct((_NB,), jnp.int32),
        jax.ShapeDtypeStruct((_T,), jnp.int32),
        jax.ShapeDtypeStruct((_T,), jnp.int32),
    ]
    scratch = [
        pltpu.VMEM((_E,), jnp.int32),
        pltpu.VMEM((_P,), jnp.int32),
        pltpu.VMEM((_P,), jnp.float32),
        pltpu.VMEM((_P,), jnp.float32),
        pltpu.VMEM((_P,), jnp.int32),
        pltpu.VMEM((_G,), jnp.int32),
        pltpu.VMEM((_G,), jnp.float32),
        pltpu.VMEM((_G,), jnp.int32),
        pltpu.VMEM((_NB,), jnp.int32),
        pltpu.SemaphoreType.DMA,
    ]

    @functools.partial(pl.kernel, mesh=mesh, out_type=out_type,
                       scratch_types=scratch,
                       compiler_params=pltpu.CompilerParams(
                           needs_layout_passes=False))
    def _route_build(poff_hbm, e0_hbm, e1_hbm, r0_hbm, r1_hbm, wa0_hbm,
                     wa1_hbm, st_hbm, was_hbm, be_hbm, p0_hbm, p1_hbm,
                     poff_v, ebuf_v, rbuf_v, wabuf_v, pos_v,
                     st_v, was_v, se_v, be_v, sem):
        wid = lax.axis_index("s") * _NC + lax.axis_index("c")

        @pl.when(wid == 0)
        def _():
            pltpu.sync_copy(poff_hbm, rbuf_v.at[pl.ds(0, _E)])
            pltpu.sync_copy(e0_hbm, ebuf_v.at[pl.ds(0, _T)])
            pltpu.sync_copy(e1_hbm, ebuf_v.at[pl.ds(_T, _T)])
            pltpu.sync_copy(wa0_hbm, wabuf_v.at[pl.ds(0, _T)])
            pltpu.sync_copy(wa1_hbm, wabuf_v.at[pl.ds(_T, _T)])

            for j in range(_E // 16):
                sl = pl.ds(j * 16, 16)
                poff_v[sl] = rbuf_v[sl].astype(jnp.int32)

            pltpu.sync_copy(r0_hbm, rbuf_v.at[pl.ds(0, _T)])
            pltpu.sync_copy(r1_hbm, rbuf_v.at[pl.ds(_T, _T)])

            def zb(i, _):
                sl = pl.ds(i * 16, 16)
                st_v[sl] = jnp.zeros((16,), jnp.int32)
                se_v[sl] = jnp.zeros((16,), jnp.int32)
                was_v[sl] = jnp.zeros((16,), jnp.float32)
                return _
            lax.fori_loop(0, _G // 16, zb, 0)

            def sb(i, _):
                base = i * 16
                sl = pl.ds(base, 16)
                ech = ebuf_v[sl]
                pof = plsc.load_gather(poff_v, [ech])
                pos = rbuf_v[sl].astype(jnp.int32) + pof
                pos_v[sl] = pos
                pvec = base + lax.iota(jnp.int32, 16)
                tok = jnp.where(pvec < _T, pvec, pvec - _T)
                plsc.store_scatter(st_v, [pos], tok)
                plsc.store_scatter(was_v, [pos], wabuf_v[sl])
                plsc.store_scatter(se_v, [pos], ech)
                return _
            lax.fori_loop(0, _P // 16, sb, 0)

            for j in range(_NB // 16):
                sbase = ((j * 16 + lax.iota(jnp.int32, 16))) * _BT
                be_v[pl.ds(j * 16, 16)] = plsc.load_gather(se_v, [sbase])

            pltpu.sync_copy(st_v, st_hbm)
            pltpu.sync_copy(was_v, was_hbm)
            pltpu.sync_copy(be_v, be_hbm)
            pltpu.sync_copy(pos_v.at[pl.ds(0, _T)], p0_hbm)
            pltpu.sync_copy(pos_v.at[pl.ds(_T, _T)], p1_hbm)

    return _route_build


@functools.lru_cache(maxsize=None)
def _make_sort_gather():
    mesh = plsc.VectorSubcoreMesh(core_axis_name="c", subcore_axis_name="s", num_cores=_NC, num_subcores=_NS)
    rows_per_w = _G // _NW
    chunk = 64
    nr = rows_per_w // chunk
    hw = _D // 2
    scratch = [
        pltpu.VMEM((rows_per_w,), jnp.int32),
        pltpu.VMEM((chunk, hw), jnp.int32),
        pltpu.VMEM((chunk, hw), jnp.int32),
        pltpu.SemaphoreType.DMA,
        pltpu.SemaphoreType.DMA,
        pltpu.SemaphoreType.DMA,
        pltpu.SemaphoreType.DMA,
    ]

    @functools.partial(
        pl.kernel, mesh=mesh,
        out_type=jax.ShapeDtypeStruct((_G, _D // 2), jnp.int32),
        scratch_types=scratch,
        compiler_params=pltpu.CompilerParams(needs_layout_passes=False))
    def _sort_gather(x_hbm, st_hbm, out_hbm, idx_v, rows0_v, rows1_v,
                     g0_sem, g1_sem, w0_sem, w1_sem):
        wid = lax.axis_index("s") * _NC + lax.axis_index("c")
        base = wid * rows_per_w
        pltpu.sync_copy(st_hbm.at[pl.ds(base, rows_per_w)], idx_v)
        rows = [rows0_v, rows1_v]
        gsem = [g0_sem, g1_sem]
        wsem = [w0_sem, w1_sem]
        gcp = [None, None]
        wcp = [None, None]
        gcp[0] = pltpu.async_copy(
            x_hbm.at[idx_v.at[pl.ds(0, chunk)]], rows[0], gsem[0])
        for r in range(nr):
            i = r % 2
            ni = (r + 1) % 2
            if r + 1 < nr:
                if wcp[ni] is not None:
                    wcp[ni].wait()
                gcp[ni] = pltpu.async_copy(
                    x_hbm.at[idx_v.at[pl.ds((r + 1) * chunk, chunk)]],
                    rows[ni], gsem[ni])
            gcp[i].wait()
            wcp[i] = pltpu.async_copy(
                rows[i], out_hbm.at[pl.ds(base + r * chunk, chunk)], wsem[i])
        wcp[0].wait()
        wcp[1].wait()

    return _sort_gather


def _expert_body(be_ref, x_ref, gw_ref, uw_ref, dw_ref, wa_ref, y_ref):
    x = x_ref[...].astype(jnp.float32)
    g = lax.dot_general(x, gw_ref[0], (((1,), (1,)), ((), ())),
                        preferred_element_type=jnp.float32)
    u = lax.dot_general(x, uw_ref[0], (((1,), (1,)), ((), ())),
                        preferred_element_type=jnp.float32)
    h = g * jax.nn.sigmoid(g) * u
    y = lax.dot_general(h, dw_ref[0], (((1,), (1,)), ((), ())),
                        preferred_element_type=jnp.float32)
    y_ref[...] = y * wa_ref[0, 0, :][:, None]


def _run_expert(be, x_sorted, egw, euw, edw, was3):
    grid_spec = pltpu.PrefetchScalarGridSpec(
        num_scalar_prefetch=1,
        grid=(_NB,),
        in_specs=[
            pl.BlockSpec((_BT, _D), lambda i, be: (i, 0)),
            pl.BlockSpec((1, _F, _D), lambda i, be: (be[i], 0, 0)),
            pl.BlockSpec((1, _F, _D), lambda i, be: (be[i], 0, 0)),
            pl.BlockSpec((1, _D, _F), lambda i, be: (be[i], 0, 0)),
            pl.BlockSpec((1, 1, _BT), lambda i, be: (i, 0, 0)),
        ],
        out_specs=pl.BlockSpec((_BT, _D), lambda i, be: (i, 0)),
    )
    return pl.pallas_call(
        _expert_body,
        grid_spec=grid_spec,
        out_shape=jax.ShapeDtypeStruct((_G, _D), jnp.float32),
    )(be, x_sorted, egw, euw, edw, was3)


@functools.lru_cache(maxsize=None)
def _make_combine():
    mesh = plsc.VectorSubcoreMesh(core_axis_name="c", subcore_axis_name="s", num_cores=_NC, num_subcores=_NS)
    toks_per_w = _T // _NW
    ch = 16
    scratch = [
        pltpu.VMEM((ch,), jnp.int32),
        pltpu.VMEM((ch,), jnp.int32),
        pltpu.VMEM((ch, _D), jnp.float32),
        pltpu.VMEM((ch, _D), jnp.float32),
        pltpu.VMEM((ch, _D), jnp.float32),
        pltpu.SemaphoreType.DMA,
    ]

    @functools.partial(
        pl.kernel, mesh=mesh,
        out_type=jax.ShapeDtypeStruct((_T, _D), jnp.float32),
        scratch_types=scratch,
        compiler_params=pltpu.CompilerParams(needs_layout_passes=False))
    def _combine(sh_hbm, y_hbm, p0_hbm, p1_hbm, out_hbm,
                 i0_v, i1_v, acc_v, r0_v, r1_v, sem):
        wid = lax.axis_index("s") * _NC + lax.axis_index("c")
        base = wid * toks_per_w

        def rb(r, _):
            tb = base + r * ch
            pltpu.sync_copy(sh_hbm.at[pl.ds(tb, ch)], acc_v)
            pltpu.sync_copy(p0_hbm.at[pl.ds(tb, ch)], i0_v)
            pltpu.sync_copy(p1_hbm.at[pl.ds(tb, ch)], i1_v)
            pltpu.async_copy(y_hbm.at[i0_v], r0_v, sem).wait()
            pltpu.async_copy(y_hbm.at[i1_v], r1_v, sem).wait()

            def tb_loop(t, _):
                def jb(j, _):
                    sl = pl.ds(j * 16, 16)
                    acc_v[t, sl] = acc_v[t, sl] + r0_v[t, sl] + r1_v[t, sl]
                    return _
                lax.fori_loop(0, _D // 16, jb, 0)
                return _
            lax.fori_loop(0, ch, tb_loop, 0)
            pltpu.sync_copy(acc_v, out_hbm.at[pl.ds(tb, ch)])
            return _
        lax.fori_loop(0, toks_per_w // ch, rb, 0)

    return _combine


def kernel(x, router_weight, shared_gate_w, shared_up_w, shared_down_w,
           expert_gate_w, expert_up_w, expert_down_w, alpha):
    x_flat = x.reshape(_T, _D)
    xb = x_flat.astype(jnp.bfloat16)
    alpha2 = alpha.reshape(1, _E)

    e0, e1, wa0, wa1, r0, r1, c, poff = _run_router(
        x_flat, router_weight, alpha2)

    st, was, be, p0, p1 = _make_route_build()(
        poff.reshape(_E), e0.reshape(_T), e1.reshape(_T),
        r0.reshape(_T), r1.reshape(_T), wa0.reshape(_T), wa1.reshape(_T))

    shared_c = _run_shared(xb, shared_gate_w.astype(jnp.bfloat16),
                           shared_up_w.astype(jnp.bfloat16),
                           shared_down_w.astype(jnp.bfloat16),
                           c.reshape(_T // _TA, 1, _TA))

    xb32 = lax.bitcast_convert_type(xb.reshape(_T, _D // 2, 2), jnp.int32)
    xs32 = _make_sort_gather()(xb32, st)
    x_sorted = lax.bitcast_convert_type(xs32, jnp.bfloat16).reshape(_G, _D)

    y = _run_expert(be, x_sorted, expert_gate_w, expert_up_w, expert_down_w,
                    was.reshape(_NB, 1, _BT))

    out = _make_combine()(shared_c, y, p0, p1)
    return out.reshape(_B, _S, _D)


# speedup vs baseline: 2.2845x; 2.2845x over previous
import functools

import jax
import jax.numpy as jnp
from jax import lax
from jax.experimental import pallas as pl
from jax.experimental.pallas import tpu as pltpu
from jax.experimental.pallas import tpu_sc as plsc

_B, _S, _D, _F, _E, _K = 2, 2048, 1024, 1024, 64, 2
_T = _B * _S
_P = _T * _K
_BT = 256
_NB = _P // _BT + _E
_G = _NB * _BT
_NCH = _G // 16
_TB = 256
_TA = 256

_NC, _NS = 2, 16
_NW = _NC * _NS


def _router_body(x_ref, rw_ref, alpha_ref, e0_ref, e1_ref, wa0_ref, wa1_ref,
                 r0_ref, r1_ref, c_ref, poff_ref, be_ref, creal_ref,
                 crank_ref, nw_ref, run_ref):
    i = pl.program_id(0)

    @pl.when(i == 0)
    def _():
        run_ref[...] = jnp.zeros_like(run_ref)

    x = x_ref[...]
    logits = lax.dot_general(x, rw_ref[...], (((1,), (1,)), ((), ())),
                             preferred_element_type=jnp.float32)
    eidx = lax.broadcasted_iota(jnp.int32, (_TB, _E), 1)
    m1 = jnp.max(logits, axis=1)
    a1 = jnp.argmax(logits, axis=1).astype(jnp.int32)
    masked = jnp.where(eidx == a1[:, None], -1e30, logits)
    m2 = jnp.max(masked, axis=1)
    a2 = jnp.argmax(masked, axis=1).astype(jnp.int32)
    w1 = 1.0 / (1.0 + jnp.exp(m2 - m1))
    w2 = 1.0 - w1

    oh1 = (eidx == a1[:, None]).astype(jnp.float32)
    oh2 = (eidx == a2[:, None]).astype(jnp.float32)
    al = alpha_ref[0, :]
    as1 = jnp.sum(oh1 * al[None, :], axis=1)
    as2 = jnp.sum(oh2 * al[None, :], axis=1)
    c = 1.0 - (w1 * as1 + w2 * as2)

    oh = jnp.concatenate([oh1, oh2], axis=0)
    t = oh
    for d in (1, 2, 4, 8, 16, 32, 64, 128, 256):
        t = t + jnp.concatenate(
            [jnp.zeros((d, _E), jnp.float32), t[:-d]], axis=0)
    ex = t - oh
    run = run_ref[...]
    rank = ex + run
    rank_vec = jnp.sum(rank * oh, axis=1)
    run_new = run + t[2 * _TB - 1:2 * _TB, :]
    run_ref[...] = run_new

    pad = jnp.floor((run_new + (_BT - 1)) * (1.0 / _BT)) * _BT
    pt = pad
    for d in (1, 2, 4, 8, 16, 32):
        pt = pt + jnp.concatenate(
            [jnp.zeros((1, d), jnp.float32), pt[:, :_E - d]], axis=1)
    poff = pt - pad
    poff_ref[...] = poff

    @pl.when(i == _T // _TB - 1)
    def _():
        tstart = (lax.broadcasted_iota(jnp.int32, (_NB, _E), 0)
                  * _BT).astype(jnp.float32)
        be_ref[...] = (jnp.sum((poff <= tstart).astype(jnp.float32), axis=1)
                       - 1.0).reshape(1, _NB)

        csf = (lax.broadcasted_iota(jnp.int32, (_NCH, 1), 0)
               * 16).astype(jnp.float32)
        ee = lax.broadcasted_iota(
            jnp.int32, (_NCH, _E), 1).astype(jnp.float32)
        e_ch = jnp.sum((poff <= csf).astype(jnp.float32), axis=1) - 1.0
        ohc = (ee == e_ch[:, None]).astype(jnp.float32)
        thr = jnp.sum(ohc * (poff + run_new), axis=1)
        real = (csf[:, 0] < thr).astype(jnp.float32)[:, None]
        tt = real
        for d in (1, 2, 4, 8, 16, 32, 64, 128, 256, 512, 1024):
            tt = tt + jnp.concatenate(
                [jnp.zeros((d, 1), jnp.float32), tt[:-d]], axis=0)
        creal_ref[...] = real
        crank_ref[...] = tt - real
        n_l = tt[_NCH - 1:_NCH, :]
        wv = lax.broadcasted_iota(
            jnp.int32, (1, _NW), 1).astype(jnp.float32)
        nw_ref[...] = jnp.maximum(
            jnp.floor((n_l - wv + (_NW - 1)) * (1.0 / _NW)), 0.0)

    e0_ref[...] = a1.reshape(1, 1, _TB)
    e1_ref[...] = a2.reshape(1, 1, _TB)
    wa0_ref[...] = (w1 * as1).reshape(1, 1, _TB)
    wa1_ref[...] = (w2 * as2).reshape(1, 1, _TB)
    r0_ref[...] = rank_vec[:_TB].reshape(1, 1, _TB)
    r1_ref[...] = rank_vec[_TB:].reshape(1, 1, _TB)
    c_ref[...] = c.reshape(1, 1, _TB)


def _run_router(x_flat, router_weight, alpha2):
    n = _T // _TB
    out3 = jax.ShapeDtypeStruct((n, 1, _TB), jnp.float32)
    out3i = jax.ShapeDtypeStruct((n, 1, _TB), jnp.int32)
    spec3 = pl.BlockSpec((1, 1, _TB), lambda i: (i, 0, 0))
    return pl.pallas_call(
        _router_body,
        grid=(n,),
        in_specs=[
            pl.BlockSpec((_TB, _D), lambda i: (i, 0)),
            pl.BlockSpec((_E, _D), lambda i: (0, 0)),
            pl.BlockSpec((1, _E), lambda i: (0, 0)),
        ],
        out_specs=[spec3, spec3, spec3, spec3, spec3, spec3, spec3,
                   pl.BlockSpec((1, _E), lambda i: (0, 0)),
                   pl.BlockSpec((1, _NB), lambda i: (0, 0)),
                   pl.BlockSpec((_NCH, 1), lambda i: (0, 0)),
                   pl.BlockSpec((_NCH, 1), lambda i: (0, 0)),
                   pl.BlockSpec((1, _NW), lambda i: (0, 0))],
        out_shape=[out3i, out3i, out3, out3, out3, out3, out3,
                   jax.ShapeDtypeStruct((1, _E), jnp.float32),
                   jax.ShapeDtypeStruct((1, _NB), jnp.float32),
                   jax.ShapeDtypeStruct((_NCH, 1), jnp.float32),
                   jax.ShapeDtypeStruct((_NCH, 1), jnp.float32),
                   jax.ShapeDtypeStruct((1, _NW), jnp.float32)],
        scratch_shapes=[pltpu.VMEM((1, _E), jnp.float32)],
    )(x_flat, router_weight, alpha2)


def _shared_body(x_ref, gw_ref, uw_ref, dw_ref, c_ref, out_ref):
    x = x_ref[...]
    g = lax.dot_general(x, gw_ref[...], (((1,), (1,)), ((), ())),
                        preferred_element_type=jnp.float32)
    u = lax.dot_general(x, uw_ref[...], (((1,), (1,)), ((), ())),
                        preferred_element_type=jnp.float32)
    h = g * jax.nn.sigmoid(g) * u
    y = lax.dot_general(h, dw_ref[...], (((1,), (1,)), ((), ())),
                        preferred_element_type=jnp.float32)
    out_ref[...] = y * c_ref[0, 0, :][:, None]


def _run_shared(x_flat, sgw, suw, sdw, c3):
    n = _T // _TA
    return pl.pallas_call(
        _shared_body,
        grid=(n,),
        in_specs=[
            pl.BlockSpec((_TA, _D), lambda i: (i, 0)),
            pl.BlockSpec((_F, _D), lambda i: (0, 0)),
            pl.BlockSpec((_F, _D), lambda i: (0, 0)),
            pl.BlockSpec((_D, _F), lambda i: (0, 0)),
            pl.BlockSpec((1, 1, _TA), lambda i: (i, 0, 0)),
        ],
        out_specs=pl.BlockSpec((_TA, _D), lambda i: (i, 0)),
        out_shape=jax.ShapeDtypeStruct((_T, _D), jnp.float32),
    )(x_flat, sgw, suw, sdw, c3)


@functools.lru_cache(maxsize=None)
def _make_route_build():
    mesh = plsc.VectorSubcoreMesh(core_axis_name="c", subcore_axis_name="s", num_cores=_NC, num_subcores=_NS)
    out_type = [
        jax.ShapeDtypeStruct((_G,), jnp.int32),
        jax.ShapeDtypeStruct((_G,), jnp.float32),
        jax.ShapeDtypeStruct((_T,), jnp.int32),
        jax.ShapeDtypeStruct((_T,), jnp.int32),
        jax.ShapeDtypeStruct((_NCH,), jnp.int32),
    ]
    scratch = [
        pltpu.VMEM((_E,), jnp.int32),
        pltpu.VMEM((_P,), jnp.int32),
        pltpu.VMEM((_P,), jnp.float32),
        pltpu.VMEM((_P,), jnp.float32),
        pltpu.VMEM((_P,), jnp.int32),
        pltpu.VMEM((_G,), jnp.int32),
        pltpu.VMEM((_G,), jnp.float32),
        pltpu.VMEM((_NCH,), jnp.int32),
        pltpu.VMEM((_NCH,), jnp.int32),
        pltpu.VMEM((_NCH,), jnp.int32),
        pltpu.VMEM((_NCH,), jnp.int32),
        pltpu.SemaphoreType.DMA,
    ]

    @functools.partial(pl.kernel, mesh=mesh, out_type=out_type,
                       scratch_types=scratch,
                       compiler_params=pltpu.CompilerParams(
                           needs_layout_passes=False))
    def _route_build(poff_hbm, e0_hbm, e1_hbm, r0_hbm, r1_hbm, wa0_hbm,
                     wa1_hbm, creal_hbm, crank_hbm,
                     st_hbm, was_hbm, p0_hbm, p1_hbm, listw_hbm,
                     poff_v, ebuf_v, rbuf_v, wabuf_v, pos_v,
                     st_v, was_v, creal_v, crank_v, tmp_v, listw_v, sem):
        wid = lax.axis_index("s") * _NC + lax.axis_index("c")

        @pl.when(wid == 0)
        def _():
            pltpu.sync_copy(poff_hbm, rbuf_v.at[pl.ds(0, _E)])
            pltpu.sync_copy(e0_hbm, ebuf_v.at[pl.ds(0, _T)])
            pltpu.sync_copy(e1_hbm, ebuf_v.at[pl.ds(_T, _T)])
            pltpu.sync_copy(wa0_hbm, wabuf_v.at[pl.ds(0, _T)])
            pltpu.sync_copy(wa1_hbm, wabuf_v.at[pl.ds(_T, _T)])
            pltpu.sync_copy(creal_hbm, creal_v)
            pltpu.sync_copy(crank_hbm, crank_v)

            for j in range(_E // 16):
                sl = pl.ds(j * 16, 16)
                poff_v[sl] = rbuf_v[sl].astype(jnp.int32)

            pltpu.sync_copy(r0_hbm, rbuf_v.at[pl.ds(0, _T)])
            pltpu.sync_copy(r1_hbm, rbuf_v.at[pl.ds(_T, _T)])

            def zb(i, _):
                sl = pl.ds(i * 16, 16)
                st_v[sl] = jnp.zeros((16,), jnp.int32)
                was_v[sl] = jnp.zeros((16,), jnp.float32)
                return _
            lax.fori_loop(0, _G // 16, zb, 0)

            def sb(i, _):
                base = i * 16
                sl = pl.ds(base, 16)
                ech = ebuf_v[sl]
                pof = plsc.load_gather(poff_v, [ech])
                pos = rbuf_v[sl].astype(jnp.int32) + pof
                pos_v[sl] = pos
                pvec = base + lax.iota(jnp.int32, 16)
                tok = jnp.where(pvec < _T, pvec, pvec - _T)
                plsc.store_scatter(st_v, [pos], tok)
                plsc.store_scatter(was_v, [pos], wabuf_v[sl])
                return _
            lax.fori_loop(0, _P // 16, sb, 0)

            n_l = jnp.int32(0)
            for g in range(_NCH // 16):
                sl = pl.ds(g * 16, 16)
                rl = creal_v[sl]
                jv = g * 16 + lax.iota(jnp.int32, 16)
                plsc.store_scatter(tmp_v, [crank_v[sl]], jv, mask=(rl != 0))
                n_l = n_l + jnp.sum(rl)
            per_w = _NCH // _NW
            for k in range(per_w):
                for h in range(_NW // 16):
                    wv = h * 16 + lax.iota(jnp.int32, 16)
                    idxv = wv + _NW * k
                    g = plsc.load_gather(tmp_v, [jnp.minimum(idxv, n_l - 1)])
                    lw = jnp.where(idxv < n_l, g, wv)
                    plsc.store_scatter(listw_v, [wv * per_w + k], lw)

            pltpu.sync_copy(st_v, st_hbm)
            pltpu.sync_copy(was_v, was_hbm)
            pltpu.sync_copy(listw_v, listw_hbm)
            pltpu.sync_copy(pos_v.at[pl.ds(0, _T)], p0_hbm)
            pltpu.sync_copy(pos_v.at[pl.ds(_T, _T)], p1_hbm)

    return _route_build


@functools.lru_cache(maxsize=None)
def _make_sort_gather():
    mesh = plsc.VectorSubcoreMesh(core_axis_name="c", subcore_axis_name="s", num_cores=_NC, num_subcores=_NS)
    chunk = 16
    nbuf = 4
    per_w = _NCH // _NW
    scratch = (
        [pltpu.VMEM((_G,), jnp.int32),
         pltpu.VMEM((per_w,), jnp.int32),
         pltpu.VMEM((_NW,), jnp.int32)]
        + [pltpu.VMEM((chunk, _D), jnp.float32) for _ in range(nbuf)]
        + [pltpu.SemaphoreType.DMA for _ in range(2 * nbuf)]
    )

    @functools.partial(
        pl.kernel, mesh=mesh,
        out_type=jax.ShapeDtypeStruct((_G, _D), jnp.float32),
        scratch_types=scratch,
        compiler_params=pltpu.CompilerParams(needs_layout_passes=False))
    def _sort_gather(x_hbm, st_hbm, listw_hbm, nw_hbm, out_hbm,
                     idx_v, lst_v, nw_v, *bufs_and_sems):
        rows = list(bufs_and_sems[:nbuf])
        gsem = list(bufs_and_sems[nbuf:2 * nbuf])
        wsem = list(bufs_and_sems[2 * nbuf:3 * nbuf])
        wid = lax.axis_index("s") * _NC + lax.axis_index("c")
        pltpu.sync_copy(st_hbm, idx_v)
        pltpu.sync_copy(listw_hbm.at[pl.ds(wid * per_w, per_w)], lst_v)
        pltpu.sync_copy(nw_hbm, nw_v)
        half = (wid // 16) * 16
        lane = wid - half
        l16 = lax.iota(jnp.int32, 16)
        n_w = jnp.sum(jnp.where(l16 == lane, nw_v[pl.ds(half, 16)], 0))
        ngrp = (n_w + (nbuf - 1)) // nbuf

        def grp(g, carry):
            cid = []
            for j in range(nbuf):
                e = g * nbuf + j
                gb = (e // 16) * 16
                vv = lst_v[pl.ds(gb, 16)]
                cid.append(jnp.sum(jnp.where(l16 == (e - gb), vv, 0)))

            @pl.when(g > 0)
            def _():
                for j in range(nbuf):
                    pltpu.make_async_copy(
                        rows[j], out_hbm.at[pl.ds(0, chunk)], wsem[j]).wait()
            gcp = []
            for j in range(nbuf):
                gcp.append(pltpu.async_copy(
                    x_hbm.at[idx_v.at[pl.ds(cid[j] * chunk, chunk)]],
                    rows[j], gsem[j]))
            for j in range(nbuf):
                gcp[j].wait()
                pltpu.async_copy(
                    rows[j], out_hbm.at[pl.ds(cid[j] * chunk, chunk)],
                    wsem[j])
            return carry
        lax.fori_loop(0, ngrp, grp, 0)

        @pl.when(ngrp > 0)
        def _():
            for j in range(nbuf):
                pltpu.make_async_copy(
                    rows[j], out_hbm.at[pl.ds(0, chunk)], wsem[j]).wait()

    return _sort_gather


def _expert_body(be_ref, x_ref, gw_ref, uw_ref, dw_ref, wa_ref, y_ref):
    x = x_ref[...]
    g = lax.dot_general(x, gw_ref[0], (((1,), (1,)), ((), ())),
                        preferred_element_type=jnp.float32)
    u = lax.dot_general(x, uw_ref[0], (((1,), (1,)), ((), ())),
                        preferred_element_type=jnp.float32)
    h = g * jax.nn.sigmoid(g) * u
    y = lax.dot_general(h, dw_ref[0], (((1,), (1,)), ((), ())),
                        preferred_element_type=jnp.float32)
    y_ref[...] = y * wa_ref[0, 0, :][:, None]


def _run_expert(be, x_sorted, egw, euw, edw, was3):
    grid_spec = pltpu.PrefetchScalarGridSpec(
        num_scalar_prefetch=1,
        grid=(_NB,),
        in_specs=[
            pl.BlockSpec((_BT, _D), lambda i, be: (i, 0)),
            pl.BlockSpec((1, _F, _D), lambda i, be: (be[i], 0, 0)),
            pl.BlockSpec((1, _F, _D), lambda i, be: (be[i], 0, 0)),
            pl.BlockSpec((1, _D, _F), lambda i, be: (be[i], 0, 0)),
            pl.BlockSpec((1, 1, _BT), lambda i, be: (i, 0, 0)),
        ],
        out_specs=pl.BlockSpec((_BT, _D), lambda i, be: (i, 0)),
    )
    return pl.pallas_call(
        _expert_body,
        grid_spec=grid_spec,
        out_shape=jax.ShapeDtypeStruct((_G, _D), jnp.float32),
    )(be, x_sorted, egw, euw, edw, was3)


@functools.lru_cache(maxsize=None)
def _make_combine():
    mesh = plsc.VectorSubcoreMesh(core_axis_name="c", subcore_axis_name="s", num_cores=_NC, num_subcores=_NS)
    toks_per_w = _T // _NW
    ch = 16
    nr = toks_per_w // ch
    scratch = [
        pltpu.VMEM((toks_per_w,), jnp.int32),
        pltpu.VMEM((toks_per_w,), jnp.int32),
        pltpu.VMEM((ch, _D), jnp.float32),
        pltpu.VMEM((ch, _D), jnp.float32),
        pltpu.VMEM((ch, _D), jnp.float32),
        pltpu.VMEM((ch, _D), jnp.float32),
        pltpu.VMEM((ch, _D), jnp.float32),
        pltpu.SemaphoreType.DMA,
        pltpu.SemaphoreType.DMA,
        pltpu.SemaphoreType.DMA,
        pltpu.SemaphoreType.DMA,
        pltpu.SemaphoreType.DMA,
    ]

    @functools.partial(
        pl.kernel, mesh=mesh,
        out_type=jax.ShapeDtypeStruct((_T, _D), jnp.float32),
        scratch_types=scratch,
        compiler_params=pltpu.CompilerParams(needs_layout_passes=False))
    def _combine(sh_hbm, y_hbm, p0_hbm, p1_hbm, out_hbm,
                 i0_v, i1_v, acc0_v, acc1_v, r00_v, r01_v, r1_v,
                 s0_sem, s1_sem, g0_sem, g1_sem, w_sem):
        wid = lax.axis_index("s") * _NC + lax.axis_index("c")
        base = wid * toks_per_w
        pltpu.sync_copy(p0_hbm.at[pl.ds(base, toks_per_w)], i0_v)
        pltpu.sync_copy(p1_hbm.at[pl.ds(base, toks_per_w)], i1_v)
        accs = [acc0_v, acc1_v]
        r0s = [r00_v, r01_v]
        ssem = [s0_sem, s1_sem]
        gsem = [g0_sem, g1_sem]
        scp = [None, None]
        gcp = [None, None]
        wcp = [None, None]

        def issue(r, i):
            tb = base + r * ch
            scp[i] = pltpu.async_copy(
                sh_hbm.at[pl.ds(tb, ch)], accs[i], ssem[i])
            gcp[i] = pltpu.async_copy(
                y_hbm.at[i0_v.at[pl.ds(r * ch, ch)]], r0s[i], gsem[i])

        issue(0, 0)
        for r in range(nr):
            i = r % 2
            ni = (r + 1) % 2
            if r + 1 < nr:
                if wcp[ni] is not None:
                    wcp[ni].wait()
                issue(r + 1, ni)
            scp[i].wait()
            gcp[i].wait()
            pltpu.async_copy(
                y_hbm.at[i1_v.at[pl.ds(r * ch, ch)]], r1_v, gsem[i]).wait()

            def t_loop(t, _):
                def j_loop(j, _2):
                    sl = pl.ds(j * 16, 16)
                    accs[i][t, sl] = (accs[i][t, sl] + r0s[i][t, sl]
                                      + r1_v[t, sl])
                    return _2
                lax.fori_loop(0, _D // 16, j_loop, 0)
                return _
            lax.fori_loop(0, ch, t_loop, 0)
            wcp[i] = pltpu.async_copy(
                accs[i], out_hbm.at[pl.ds(base + r * ch, ch)], w_sem)
        wcp[0].wait()
        wcp[1].wait()

    return _combine


def kernel(x, router_weight, shared_gate_w, shared_up_w, shared_down_w,
           expert_gate_w, expert_up_w, expert_down_w, alpha):
    x_flat = x.reshape(_T, _D)
    alpha2 = alpha.reshape(1, _E)

    (e0, e1, wa0, wa1, r0, r1, c, poff, be_f, creal, crank,
     nw_f) = _run_router(x_flat, router_weight, alpha2)

    st, was, p0, p1, listw = _make_route_build()(
        poff.reshape(_E), e0.reshape(_T), e1.reshape(_T),
        r0.reshape(_T), r1.reshape(_T), wa0.reshape(_T), wa1.reshape(_T),
        creal.reshape(_NCH).astype(jnp.int32),
        crank.reshape(_NCH).astype(jnp.int32))

    shared_c = _run_shared(x_flat, shared_gate_w, shared_up_w, shared_down_w,
                           c.reshape(_T // _TA, 1, _TA))

    x_sorted = _make_sort_gather()(
        x_flat, st, listw, nw_f.reshape(_NW).astype(jnp.int32))

    y = _run_expert(be_f.reshape(_NB).astype(jnp.int32), x_sorted,
                    expert_gate_w, expert_up_w, expert_down_w,
                    was.reshape(_NB, 1, _BT))

    out = _make_combine()(shared_c, y, p0, p1)
    return out.reshape(_B, _S, _D)
